# Initial kernel scaffold; baseline (speedup 1.0000x reference)
import jax
import jax.numpy as jnp
from jax.experimental import pallas as pl


def kernel(graph, edge_index, rates, W1, b1, We1, be1, We2, be2, W2, b2, W3, b3, Wh, bh, Wh2, bh2, Wf, bf):
    raise NotImplementedError("write your pallas kernel here")


# speedup vs baseline: 12.1528x; 12.1528x over previous
import functools

import jax
import jax.numpy as jnp
from jax import lax
from jax.experimental import pallas as pl
from jax.experimental.pallas import tpu as pltpu
from jax.experimental.pallas import tpu_sc as plsc

f32 = jnp.float32
i32 = jnp.int32

NN = 10000
NP_ = 10240
EE = 320000
NWK = 32
KC = 128
EPW = 10240
EPAD = NWK * EPW
NCH = EPW // KC
RPT = NP_ // 16
HID = 128

def _sc_mesh():
    return plsc.VectorSubcoreMesh(core_axis_name="c", subcore_axis_name="s",
                                  num_cores=2, num_subcores=16)


def _fill(ref, n, val):
    def body(i, _):
        ref[pl.ds(i * 16, 16)] = jnp.full((16,), val, f32)
        return 0
    lax.fori_loop(0, n // 16, body, 0)


def _deg_body(dst_hbm, out_hbm, ones_v, idx_v, tmp_v, acc_sh):
    c = lax.axis_index("c")
    s = lax.axis_index("s")
    wid = c * 16 + s
    _fill(ones_v, KC, 1.0)
    _fill(tmp_v, RPT, 0.0)
    pltpu.sync_copy(tmp_v, acc_sh.at[pl.ds(s * RPT, RPT)])
    plsc.subcore_barrier()

    base = wid * EPW

    def body(j, _):
        pltpu.sync_copy(dst_hbm.at[pl.ds(base + j * KC, KC)], idx_v)
        pltpu.sync_copy(ones_v, acc_sh.at[idx_v], add=True)
        return 0
    lax.fori_loop(0, NCH, body, 0)

    plsc.subcore_barrier()
    pltpu.sync_copy(acc_sh.at[pl.ds(s * RPT, RPT)], tmp_v)
    pltpu.sync_copy(tmp_v, out_hbm.at[pl.ds(c * NP_ + s * RPT, RPT)])


def _deg_kernel(dst):
    return pl.kernel(
        _deg_body,
        out_type=jax.ShapeDtypeStruct((2 * NP_,), f32),
        mesh=_sc_mesh(),
        scratch_types=[
            pltpu.VMEM((KC,), f32),
            pltpu.VMEM((KC,), i32),
            pltpu.VMEM((RPT,), f32),
            pltpu.VMEM_SHARED((NP_,), f32),
        ],
    )(dst)


def _conv1_body(src_hbm, dst_hbm, h0p_hbm, dinv_hbm, s1_hbm, ca_hbm,
                idx_s, idx_d, rows_v, aval_v, tmp_v,
                acc_sh, cacc_sh, sem):
    c = lax.axis_index("c")
    s = lax.axis_index("s")
    wid = c * 16 + s

    def zrow(i, _):
        def zcol(j, _):
            rows_v[i, pl.ds(j * 16, 16)] = jnp.zeros((16,), f32)
            return 0
        lax.fori_loop(0, HID // 16, zcol, 0)
        return 0
    lax.fori_loop(0, KC, zrow, 0)
    _fill(tmp_v, RPT, 0.0)

    def zacc(k, _):
        pltpu.sync_copy(rows_v, acc_sh.at[pl.ds(s * RPT + k * KC, KC)])
        return 0
    lax.fori_loop(0, RPT // KC, zacc, 0)
    pltpu.sync_copy(tmp_v, cacc_sh.at[pl.ds(s * RPT, RPT)])
    plsc.subcore_barrier()

    base = wid * EPW

    def body(j, _):
        pltpu.sync_copy(src_hbm.at[pl.ds(base + j * KC, KC)], idx_s)
        pltpu.sync_copy(dst_hbm.at[pl.ds(base + j * KC, KC)], idx_d)
        pltpu.async_copy(h0p_hbm.at[idx_s], rows_v, sem).wait()
        pltpu.sync_copy(rows_v, acc_sh.at[idx_d], add=True)
        pltpu.async_copy(dinv_hbm.at[idx_d], aval_v, sem).wait()
        pltpu.sync_copy(aval_v, cacc_sh.at[idx_s], add=True)
        return 0
    lax.fori_loop(0, NCH, body, 0)

    plsc.subcore_barrier()

    def cout(k, _):
        pltpu.sync_copy(acc_sh.at[pl.ds(s * RPT + k * KC, KC)], rows_v)
        pltpu.sync_copy(rows_v, s1_hbm.at[pl.ds(c * NP_ + s * RPT + k * KC, KC)])
        return 0
    lax.fori_loop(0, RPT // KC, cout, 0)
    pltpu.sync_copy(cacc_sh.at[pl.ds(s * RPT, RPT)], tmp_v)
    pltpu.sync_copy(tmp_v, ca_hbm.at[pl.ds(c * NP_ + s * RPT, RPT)])


def _conv1_kernel(src, dst, h0p, dinv):
    return pl.kernel(
        _conv1_body,
        out_type=[
            jax.ShapeDtypeStruct((2 * NP_, HID), f32),
            jax.ShapeDtypeStruct((2 * NP_,), f32),
        ],
        mesh=_sc_mesh(),
        scratch_types=[
            pltpu.VMEM((KC,), i32),
            pltpu.VMEM((KC,), i32),
            pltpu.VMEM((KC, HID), f32),
            pltpu.VMEM((KC,), f32),
            pltpu.VMEM((RPT,), f32),
            pltpu.VMEM_SHARED((NP_, HID), f32),
            pltpu.VMEM_SHARED((NP_,), f32),
            pltpu.SemaphoreType.DMA,
        ],
    )(src, dst, h0p, dinv)


def _passb_body(src_hbm, dst_hbm, u_hbm, out_hbm,
                idx_s, idx_d, aval_v, tmp_v, acc_sh, sem):
    c = lax.axis_index("c")
    s = lax.axis_index("s")
    wid = c * 16 + s
    _fill(tmp_v, RPT, 0.0)
    pltpu.sync_copy(tmp_v, acc_sh.at[pl.ds(s * RPT, RPT)])
    plsc.subcore_barrier()

    base = wid * EPW

    def body(j, _):
        pltpu.sync_copy(src_hbm.at[pl.ds(base + j * KC, KC)], idx_s)
        pltpu.sync_copy(dst_hbm.at[pl.ds(base + j * KC, KC)], idx_d)
        pltpu.async_copy(u_hbm.at[idx_d], aval_v, sem).wait()
        pltpu.sync_copy(aval_v, acc_sh.at[idx_s], add=True)
        return 0
    lax.fori_loop(0, NCH, body, 0)

    plsc.subcore_barrier()
    pltpu.sync_copy(acc_sh.at[pl.ds(s * RPT, RPT)], tmp_v)
    pltpu.sync_copy(tmp_v, out_hbm.at[pl.ds(c * NP_ + s * RPT, RPT)])


def _passb_kernel(src, dst, u):
    return pl.kernel(
        _passb_body,
        out_type=jax.ShapeDtypeStruct((2 * NP_,), f32),
        mesh=_sc_mesh(),
        scratch_types=[
            pltpu.VMEM((KC,), i32),
            pltpu.VMEM((KC,), i32),
            pltpu.VMEM((KC,), f32),
            pltpu.VMEM((RPT,), f32),
            pltpu.VMEM_SHARED((NP_,), f32),
            pltpu.SemaphoreType.DMA,
        ],
    )(src, dst, u)


BR = 512
GRID = NP_ // BR


def _tc1_body(x_ref, w1_ref, dega_ref, degb_ref, h0p_ref, dinv_ref):
    deg = dega_ref[0] + degb_ref[0] + 1.0
    dinv = lax.rsqrt(deg)
    h = jnp.dot(x_ref[...], w1_ref[...], preferred_element_type=f32)
    h0p_ref[...] = h * dinv
    dinv_ref[...] = dinv


def _tc1(x, w1, degp):
    return pl.pallas_call(
        _tc1_body,
        grid=(GRID,),
        in_specs=[
            pl.BlockSpec((BR, HID), lambda i: (i, 0)),
            pl.BlockSpec((HID, HID), lambda i: (0, 0)),
            pl.BlockSpec((1, BR, 1), lambda i: (0, i, 0)),
            pl.BlockSpec((1, BR, 1), lambda i: (1, i, 0)),
        ],
        out_specs=[
            pl.BlockSpec((BR, HID), lambda i: (i, 0)),
            pl.BlockSpec((BR, 1), lambda i: (i, 0)),
        ],
        out_shape=[
            jax.ShapeDtypeStruct((NP_, HID), f32),
            jax.ShapeDtypeStruct((NP_, 1), f32),
        ],
    )(x, w1, degp, degp)


def _tc2_body(s1a_ref, s1b_ref, h0p_ref, dinv_ref, ca_ref, cb_ref, b1_ref,
              g1_ref, u_ref, w_ref):
    i = pl.program_id(0)
    rows = lax.broadcasted_iota(i32, (BR, 1), 0) + i * BR
    m = (rows < NN).astype(f32)
    dinv = dinv_ref[...]
    s1 = s1a_ref[0] + s1b_ref[0]
    g1 = jnp.maximum(dinv * (s1 + h0p_ref[...]) + b1_ref[...], 0.0)
    g1_ref[...] = g1 * m
    cc = ca_ref[0] + cb_ref[0]
    w = (cc + dinv) * dinv * m
    w_ref[...] = w
    u_ref[...] = w * dinv


def _tc2(s1p, h0p, dinv, cp, b1):
    return pl.pallas_call(
        _tc2_body,
        grid=(GRID,),
        in_specs=[
            pl.BlockSpec((1, BR, HID), lambda i: (0, i, 0)),
            pl.BlockSpec((1, BR, HID), lambda i: (1, i, 0)),
            pl.BlockSpec((BR, HID), lambda i: (i, 0)),
            pl.BlockSpec((BR, 1), lambda i: (i, 0)),
            pl.BlockSpec((1, BR, 1), lambda i: (0, i, 0)),
            pl.BlockSpec((1, BR, 1), lambda i: (1, i, 0)),
            pl.BlockSpec((1, HID), lambda i: (0, 0)),
        ],
        out_specs=[
            pl.BlockSpec((BR, HID), lambda i: (i, 0)),
            pl.BlockSpec((BR, 1), lambda i: (i, 0)),
            pl.BlockSpec((BR, 1), lambda i: (i, 0)),
        ],
        out_shape=[
            jax.ShapeDtypeStruct((NP_, HID), f32),
            jax.ShapeDtypeStruct((NP_, 1), f32),
            jax.ShapeDtypeStruct((NP_, 1), f32),
        ],
    )(s1p, s1p, h0p, dinv, cp, cp, b1)


def _tc3_body(g1_ref, u_ref, va_ref, vb_ref, dinv_ref, w_ref,
              rates_ref, we1_ref, be1_ref, we2_ref, be2_ref,
              w2a_ref, w2b_ref, b2_ref, w3_ref, b3_ref,
              wh_ref, bh_ref, wh2_ref, bh2_ref, wf_ref, bf_ref,
              out_ref, tacc, sacc):
    i = pl.program_id(0)

    @pl.when(i == 0)
    def _():
        tacc[...] = jnp.zeros((1, HID), f32)
        sacc[0] = 0.0
        sacc[1] = 0.0

    rows = lax.broadcasted_iota(i32, (BR, 1), 0) + i * BR
    m = (rows < NN).astype(f32)
    q = (va_ref[0] + vb_ref[0] + u_ref[...]) * dinv_ref[...] * m
    tacc[...] += jnp.sum(q * g1_ref[...], axis=0, keepdims=True)
    sacc[0] += jnp.sum(q)
    sacc[1] += jnp.sum(w_ref[...])

    @pl.when(i == pl.num_programs(0) - 1)
    def _():
        dot = functools.partial(jnp.dot, preferred_element_type=f32)
        r = jnp.maximum(dot(rates_ref[...], we1_ref[...]) + be1_ref[...], 0.0)
        r = dot(r, we2_ref[...]) + be2_ref[...]
        wtg2 = (dot(tacc[...], w2a_ref[...]) + sacc[0] * dot(r, w2b_ref[...])
                + sacc[1] * b2_ref[...])
        pool = dot(wtg2, w3_ref[...]) * (1.0 / NN) + b3_ref[...]
        z = jnp.maximum(dot(pool, wh_ref[...]) + bh_ref[...], 0.0)
        z = jnp.maximum(dot(z, wh2_ref[...]) + bh2_ref[...], 0.0)
        out_ref[...] = dot(z, wf_ref[...]) + bf_ref[...]


def _tc3(g1, u, vp, dinv, w, smalls):
    full = lambda a: pl.BlockSpec(a.shape, lambda i: tuple(0 for _ in a.shape))
    return pl.pallas_call(
        _tc3_body,
        grid=(GRID,),
        in_specs=[
            pl.BlockSpec((BR, HID), lambda i: (i, 0)),
            pl.BlockSpec((BR, 1), lambda i: (i, 0)),
            pl.BlockSpec((1, BR, 1), lambda i: (0, i, 0)),
            pl.BlockSpec((1, BR, 1), lambda i: (1, i, 0)),
            pl.BlockSpec((BR, 1), lambda i: (i, 0)),
            pl.BlockSpec((BR, 1), lambda i: (i, 0)),
        ] + [full(a) for a in smalls],
        out_specs=pl.BlockSpec((1, 2), lambda i: (0, 0)),
        out_shape=jax.ShapeDtypeStruct((1, 2), f32),
        scratch_shapes=[pltpu.VMEM((1, HID), f32), pltpu.SMEM((2,), f32)],
    )(g1, u, vp, vp, dinv, w, *smalls)


def kernel(graph, edge_index, rates, W1, b1, We1, be1, We2, be2, W2, b2,
           W3, b3, Wh, bh, Wh2, bh2, Wf, bf):
    pad = EPAD - EE
    src = jnp.concatenate([edge_index[0], jnp.full((pad,), NP_ - 1, i32)])
    dst = jnp.concatenate([edge_index[1], jnp.full((pad,), NP_ - 1, i32)])
    x = jnp.pad(graph, ((0, NP_ - NN), (0, 0)))

    degp = _deg_kernel(dst)
    h0p, dinv = _tc1(x, W1, degp.reshape(2, NP_, 1))
    s1p, cp = _conv1_kernel(src, dst, h0p, dinv.reshape(NP_))
    g1, u, w = _tc2(s1p.reshape(2, NP_, HID), h0p, dinv,
                    cp.reshape(2, NP_, 1), b1.reshape(1, HID))
    vp = _passb_kernel(src, dst, u.reshape(NP_))
    smalls = (rates.reshape(1, 16), We1, be1.reshape(1, 8), We2,
              be2.reshape(1, HID), W2[:HID], W2[HID:], b2.reshape(1, HID),
              W3, b3.reshape(1, HID), Wh, bh.reshape(1, 2 * HID), Wh2,
              bh2.reshape(1, HID), Wf, bf.reshape(1, 2))
    return _tc3(g1, u, vp.reshape(2, NP_, 1), dinv, w, smalls)


# speedup vs baseline: 16.5600x; 1.3626x over previous
import functools

import jax
import jax.numpy as jnp
from jax import lax
from jax.experimental import pallas as pl
from jax.experimental.pallas import tpu as pltpu
from jax.experimental.pallas import tpu_sc as plsc

f32 = jnp.float32
i32 = jnp.int32

NN = 10000
NP_ = 10240
EE = 320000
NWK = 32
KC = 128
EPW = 10240
EPAD = NWK * EPW
NCH = EPW // KC
RPT = NP_ // 16
HID = 128

def _sc_mesh():
    return plsc.VectorSubcoreMesh(core_axis_name="c", subcore_axis_name="s",
                                  num_cores=2, num_subcores=16)


def _fill(ref, n, val):
    def body(i, _):
        ref[pl.ds(i * 16, 16)] = jnp.full((16,), val, f32)
        return 0
    lax.fori_loop(0, n // 16, body, 0)


NBUF = 4
NG = NCH // NBUF


def _deg_body(dst_hbm, out_hbm, ones_v, idx_d, tmp_v, acc_sh, *sems):
    c = lax.axis_index("c")
    s = lax.axis_index("s")
    wid = c * 16 + s
    _fill(ones_v, KC, 1.0)
    _fill(tmp_v, RPT, 0.0)
    pltpu.sync_copy(tmp_v, acc_sh.at[pl.ds(s * RPT, RPT)])
    pltpu.sync_copy(dst_hbm.at[pl.ds(wid * NCH, NCH)], idx_d)
    plsc.subcore_barrier()

    def group(g, _):
        ds_ = [pltpu.async_copy(ones_v, acc_sh.at[idx_d.at[g * NBUF + b]],
                                sems[b], add=True) for b in range(NBUF)]
        for d in ds_:
            d.wait()
        return 0
    lax.fori_loop(0, NG, group, 0)

    plsc.subcore_barrier()
    pltpu.sync_copy(acc_sh.at[pl.ds(s * RPT, RPT)], tmp_v)
    pltpu.sync_copy(tmp_v, out_hbm.at[pl.ds(c * NP_ + s * RPT, RPT)])


def _deg_kernel(dst2):
    return pl.kernel(
        _deg_body,
        out_type=jax.ShapeDtypeStruct((2 * NP_,), f32),
        mesh=_sc_mesh(),
        scratch_types=[
            pltpu.VMEM((KC,), f32),
            pltpu.VMEM((NCH, KC), i32),
            pltpu.VMEM((RPT,), f32),
            pltpu.VMEM_SHARED((NP_,), f32),
        ] + [pltpu.SemaphoreType.DMA] * NBUF,
    )(dst2)


CNB = 2
IST = 16
NST = NCH // IST


def _conv1_body(src_hbm, dst_hbm, h0p_hbm, dinv_hbm, s1_hbm, ca_hbm, *sc):
    idx_s, idx_d = sc[0], sc[1]
    rows = sc[2:2 + CNB]
    aval = sc[4:4 + CNB]
    acc_sh, cacc_sh = sc[6], sc[7]
    gsem = sc[8:8 + CNB]
    hsem = sc[10:10 + CNB]
    ssem = sc[12:12 + CNB]
    tsem = sc[14:14 + CNB]
    c = lax.axis_index("c")
    s = lax.axis_index("s")
    wid = c * 16 + s

    def zrow(i, _):
        def zcol(j, _):
            rows[0][i, pl.ds(j * 16, 16)] = jnp.zeros((16,), f32)
            return 0
        lax.fori_loop(0, HID // 16, zcol, 0)
        return 0
    lax.fori_loop(0, KC, zrow, 0)
    _fill(aval[0], KC, 0.0)

    def zacc(k, _):
        pltpu.sync_copy(rows[0], acc_sh.at[pl.ds(s * RPT + k * KC, KC)])
        pltpu.sync_copy(aval[0], cacc_sh.at[pl.ds(s * RPT + k * KC, KC)])
        return 0
    lax.fori_loop(0, RPT // KC, zacc, 0)
    plsc.subcore_barrier()

    def stage(st, _):
        base = wid * NCH + st * IST
        pltpu.sync_copy(src_hbm.at[pl.ds(base, IST)], idx_s)
        pltpu.sync_copy(dst_hbm.at[pl.ds(base, IST)], idx_d)

        def group(g, _):
            gd, hd = [], []
            for b in range(CNB):
                j = g * CNB + b
                gd.append(pltpu.async_copy(h0p_hbm.at[idx_s.at[j]], rows[b],
                                           gsem[b]))
                hd.append(pltpu.async_copy(dinv_hbm.at[idx_d.at[j]], aval[b],
                                           hsem[b]))
            sd, td = [], []
            for b in range(CNB):
                j = g * CNB + b
                gd[b].wait()
                sd.append(pltpu.async_copy(rows[b], acc_sh.at[idx_d.at[j]],
                                           ssem[b], add=True))
                hd[b].wait()
                td.append(pltpu.async_copy(aval[b], cacc_sh.at[idx_s.at[j]],
                                           tsem[b], add=True))
            for b in range(CNB):
                sd[b].wait()
                td[b].wait()
            return 0
        lax.fori_loop(0, IST // CNB, group, 0)
        return 0
    lax.fori_loop(0, NST, stage, 0)

    plsc.subcore_barrier()

    def cout(k, _):
        pltpu.sync_copy(acc_sh.at[pl.ds(s * RPT + k * KC, KC)], rows[0])
        pltpu.sync_copy(rows[0], s1_hbm.at[pl.ds(c * NP_ + s * RPT + k * KC, KC)])
        pltpu.sync_copy(cacc_sh.at[pl.ds(s * RPT + k * KC, KC)], aval[0])
        pltpu.sync_copy(aval[0], ca_hbm.at[pl.ds(c * NP_ + s * RPT + k * KC, KC)])
        return 0
    lax.fori_loop(0, RPT // KC, cout, 0)


def _conv1_kernel(src2, dst2, h0p, dinv):
    return pl.kernel(
        _conv1_body,
        out_type=[
            jax.ShapeDtypeStruct((2 * NP_, HID), f32),
            jax.ShapeDtypeStruct((2 * NP_,), f32),
        ],
        mesh=_sc_mesh(),
        scratch_types=[
            pltpu.VMEM((IST, KC), i32),
            pltpu.VMEM((IST, KC), i32),
        ] + [pltpu.VMEM((KC, HID), f32)] * CNB
          + [pltpu.VMEM((KC,), f32)] * CNB
          + [
            pltpu.VMEM_SHARED((NP_, HID), f32),
            pltpu.VMEM_SHARED((NP_,), f32),
        ] + [pltpu.SemaphoreType.DMA] * (4 * CNB),
    )(src2, dst2, h0p, dinv)


def _passb_body(src_hbm, dst_hbm, u_hbm, out_hbm, *sc):
    idx_s, idx_d = sc[0], sc[1]
    aval = sc[2:2 + NBUF]
    tmp_v, acc_sh = sc[6], sc[7]
    hsem = sc[8:12]
    tsem = sc[12:16]
    c = lax.axis_index("c")
    s = lax.axis_index("s")
    wid = c * 16 + s
    _fill(tmp_v, RPT, 0.0)
    pltpu.sync_copy(tmp_v, acc_sh.at[pl.ds(s * RPT, RPT)])
    pltpu.sync_copy(src_hbm.at[pl.ds(wid * NCH, NCH)], idx_s)
    pltpu.sync_copy(dst_hbm.at[pl.ds(wid * NCH, NCH)], idx_d)
    plsc.subcore_barrier()

    def group(g, _):
        hd = []
        for b in range(NBUF):
            j = g * NBUF + b
            hd.append(pltpu.async_copy(u_hbm.at[idx_d.at[j]], aval[b],
                                       hsem[b]))
        td = []
        for b in range(NBUF):
            j = g * NBUF + b
            hd[b].wait()
            td.append(pltpu.async_copy(aval[b], acc_sh.at[idx_s.at[j]],
                                       tsem[b], add=True))
        for b in range(NBUF):
            td[b].wait()
        return 0
    lax.fori_loop(0, NG, group, 0)

    plsc.subcore_barrier()
    pltpu.sync_copy(acc_sh.at[pl.ds(s * RPT, RPT)], tmp_v)
    pltpu.sync_copy(tmp_v, out_hbm.at[pl.ds(c * NP_ + s * RPT, RPT)])


def _passb_kernel(src2, dst2, u):
    return pl.kernel(
        _passb_body,
        out_type=jax.ShapeDtypeStruct((2 * NP_,), f32),
        mesh=_sc_mesh(),
        scratch_types=[
            pltpu.VMEM((NCH, KC), i32),
            pltpu.VMEM((NCH, KC), i32),
        ] + [pltpu.VMEM((KC,), f32)] * NBUF
          + [
            pltpu.VMEM((RPT,), f32),
            pltpu.VMEM_SHARED((NP_,), f32),
        ] + [pltpu.SemaphoreType.DMA] * (2 * NBUF),
    )(src2, dst2, u)


BR = 512
GRID = NP_ // BR


def _tc1_body(x_ref, w1_ref, dega_ref, degb_ref, h0p_ref, dinv_ref):
    deg = dega_ref[0] + degb_ref[0] + 1.0
    dinv = lax.rsqrt(deg)
    h = jnp.dot(x_ref[...], w1_ref[...], preferred_element_type=f32)
    h0p_ref[...] = h * dinv
    dinv_ref[...] = dinv


def _tc1(x, w1, degp):
    return pl.pallas_call(
        _tc1_body,
        grid=(GRID,),
        in_specs=[
            pl.BlockSpec((BR, HID), lambda i: (i, 0)),
            pl.BlockSpec((HID, HID), lambda i: (0, 0)),
            pl.BlockSpec((1, BR, 1), lambda i: (0, i, 0)),
            pl.BlockSpec((1, BR, 1), lambda i: (1, i, 0)),
        ],
        out_specs=[
            pl.BlockSpec((BR, HID), lambda i: (i, 0)),
            pl.BlockSpec((BR, 1), lambda i: (i, 0)),
        ],
        out_shape=[
            jax.ShapeDtypeStruct((NP_, HID), f32),
            jax.ShapeDtypeStruct((NP_, 1), f32),
        ],
    )(x, w1, degp, degp)


def _tc2_body(s1a_ref, s1b_ref, h0p_ref, dinv_ref, ca_ref, cb_ref, b1_ref,
              g1_ref, u_ref, w_ref):
    i = pl.program_id(0)
    rows = lax.broadcasted_iota(i32, (BR, 1), 0) + i * BR
    m = (rows < NN).astype(f32)
    dinv = dinv_ref[...]
    s1 = s1a_ref[0] + s1b_ref[0]
    g1 = jnp.maximum(dinv * (s1 + h0p_ref[...]) + b1_ref[...], 0.0)
    g1_ref[...] = g1 * m
    cc = ca_ref[0] + cb_ref[0]
    w = (cc + dinv) * dinv * m
    w_ref[...] = w
    u_ref[...] = w * dinv


def _tc2(s1p, h0p, dinv, cp, b1):
    return pl.pallas_call(
        _tc2_body,
        grid=(GRID,),
        in_specs=[
            pl.BlockSpec((1, BR, HID), lambda i: (0, i, 0)),
            pl.BlockSpec((1, BR, HID), lambda i: (1, i, 0)),
            pl.BlockSpec((BR, HID), lambda i: (i, 0)),
            pl.BlockSpec((BR, 1), lambda i: (i, 0)),
            pl.BlockSpec((1, BR, 1), lambda i: (0, i, 0)),
            pl.BlockSpec((1, BR, 1), lambda i: (1, i, 0)),
            pl.BlockSpec((1, HID), lambda i: (0, 0)),
        ],
        out_specs=[
            pl.BlockSpec((BR, HID), lambda i: (i, 0)),
            pl.BlockSpec((BR, 1), lambda i: (i, 0)),
            pl.BlockSpec((BR, 1), lambda i: (i, 0)),
        ],
        out_shape=[
            jax.ShapeDtypeStruct((NP_, HID), f32),
            jax.ShapeDtypeStruct((NP_, 1), f32),
            jax.ShapeDtypeStruct((NP_, 1), f32),
        ],
    )(s1p, s1p, h0p, dinv, cp, cp, b1)


def _tc3_body(g1_ref, u_ref, va_ref, vb_ref, dinv_ref, w_ref,
              rates_ref, we1_ref, be1_ref, we2_ref, be2_ref,
              w2a_ref, w2b_ref, b2_ref, w3_ref, b3_ref,
              wh_ref, bh_ref, wh2_ref, bh2_ref, wf_ref, bf_ref,
              out_ref, tacc, sacc):
    i = pl.program_id(0)

    @pl.when(i == 0)
    def _():
        tacc[...] = jnp.zeros((1, HID), f32)
        sacc[0] = 0.0
        sacc[1] = 0.0

    rows = lax.broadcasted_iota(i32, (BR, 1), 0) + i * BR
    m = (rows < NN).astype(f32)
    q = (va_ref[0] + vb_ref[0] + u_ref[...]) * dinv_ref[...] * m
    tacc[...] += jnp.sum(q * g1_ref[...], axis=0, keepdims=True)
    sacc[0] += jnp.sum(q)
    sacc[1] += jnp.sum(w_ref[...])

    @pl.when(i == pl.num_programs(0) - 1)
    def _():
        dot = functools.partial(jnp.dot, preferred_element_type=f32)
        r = jnp.maximum(dot(rates_ref[...], we1_ref[...]) + be1_ref[...], 0.0)
        r = dot(r, we2_ref[...]) + be2_ref[...]
        wtg2 = (dot(tacc[...], w2a_ref[...]) + sacc[0] * dot(r, w2b_ref[...])
                + sacc[1] * b2_ref[...])
        pool = dot(wtg2, w3_ref[...]) * (1.0 / NN) + b3_ref[...]
        z = jnp.maximum(dot(pool, wh_ref[...]) + bh_ref[...], 0.0)
        z = jnp.maximum(dot(z, wh2_ref[...]) + bh2_ref[...], 0.0)
        out_ref[...] = dot(z, wf_ref[...]) + bf_ref[...]


def _tc3(g1, u, vp, dinv, w, smalls):
    full = lambda a: pl.BlockSpec(a.shape, lambda i: tuple(0 for _ in a.shape))
    return pl.pallas_call(
        _tc3_body,
        grid=(GRID,),
        in_specs=[
            pl.BlockSpec((BR, HID), lambda i: (i, 0)),
            pl.BlockSpec((BR, 1), lambda i: (i, 0)),
            pl.BlockSpec((1, BR, 1), lambda i: (0, i, 0)),
            pl.BlockSpec((1, BR, 1), lambda i: (1, i, 0)),
            pl.BlockSpec((BR, 1), lambda i: (i, 0)),
            pl.BlockSpec((BR, 1), lambda i: (i, 0)),
        ] + [full(a) for a in smalls],
        out_specs=pl.BlockSpec((1, 2), lambda i: (0, 0)),
        out_shape=jax.ShapeDtypeStruct((1, 2), f32),
        scratch_shapes=[pltpu.VMEM((1, HID), f32), pltpu.SMEM((2,), f32)],
    )(g1, u, vp, vp, dinv, w, *smalls)


def kernel(graph, edge_index, rates, W1, b1, We1, be1, We2, be2, W2, b2,
           W3, b3, Wh, bh, Wh2, bh2, Wf, bf):
    pad = EPAD - EE
    src = jnp.concatenate([edge_index[0], jnp.full((pad,), NP_ - 1, i32)])
    dst = jnp.concatenate([edge_index[1], jnp.full((pad,), NP_ - 1, i32)])
    src2 = src.reshape(NWK * NCH, KC)
    dst2 = dst.reshape(NWK * NCH, KC)
    x = jnp.pad(graph, ((0, NP_ - NN), (0, 0)))

    degp = _deg_kernel(dst2)
    h0p, dinv = _tc1(x, W1, degp.reshape(2, NP_, 1))
    s1p, cp = _conv1_kernel(src2, dst2, h0p, dinv.reshape(NP_))
    g1, u, w = _tc2(s1p.reshape(2, NP_, HID), h0p, dinv,
                    cp.reshape(2, NP_, 1), b1.reshape(1, HID))
    vp = _passb_kernel(src2, dst2, u.reshape(NP_))
    smalls = (rates.reshape(1, 16), We1, be1.reshape(1, 8), We2,
              be2.reshape(1, HID), W2[:HID], W2[HID:], b2.reshape(1, HID),
              W3, b3.reshape(1, HID), Wh, bh.reshape(1, 2 * HID), Wh2,
              bh2.reshape(1, HID), Wf, bf.reshape(1, 2))
    return _tc3(g1, u, vp.reshape(2, NP_, 1), dinv, w, smalls)


# speedup vs baseline: 35.2263x; 2.1272x over previous
import functools

import jax
import jax.numpy as jnp
from jax import lax
from jax.experimental import pallas as pl
from jax.experimental.pallas import tpu as pltpu
from jax.experimental.pallas import tpu_sc as plsc

f32 = jnp.float32
i32 = jnp.int32

NN = 10000
NP_ = 10240
EE = 320000
NWK = 32
KC = 128
EPW = 10240
EPAD = NWK * EPW
NCH = EPW // KC
RPT = NP_ // 16
HID = 128

def _sc_mesh():
    return plsc.VectorSubcoreMesh(core_axis_name="c", subcore_axis_name="s",
                                  num_cores=2, num_subcores=16)


def _fill(ref, n, val):
    def body(i, _):
        ref[pl.ds(i * 16, 16)] = jnp.full((16,), val, f32)
        return 0
    lax.fori_loop(0, n // 16, body, 0)


NBUF = 4
NG = NCH // NBUF


def _deg_body(dst_hbm, out_hbm, ones_v, idx_d, tmp_v, acc_sh, *sems):
    c = lax.axis_index("c")
    s = lax.axis_index("s")
    wid = c * 16 + s
    _fill(ones_v, KC, 1.0)
    _fill(tmp_v, RPT, 0.0)
    pltpu.sync_copy(tmp_v, acc_sh.at[pl.ds(s * RPT, RPT)])
    pltpu.sync_copy(dst_hbm.at[pl.ds(wid * NCH, NCH)], idx_d)
    plsc.subcore_barrier()

    def group(g, _):
        ds_ = [pltpu.async_copy(ones_v, acc_sh.at[idx_d.at[g * NBUF + b]],
                                sems[b], add=True) for b in range(NBUF)]
        for d in ds_:
            d.wait()
        return 0
    lax.fori_loop(0, NG, group, 0)

    plsc.subcore_barrier()
    pltpu.sync_copy(acc_sh.at[pl.ds(s * RPT, RPT)], tmp_v)
    pltpu.sync_copy(tmp_v, out_hbm.at[pl.ds(c * NP_ + s * RPT, RPT)])


def _deg_kernel(dst2):
    return pl.kernel(
        _deg_body,
        out_type=jax.ShapeDtypeStruct((2 * NP_,), f32),
        mesh=_sc_mesh(),
        scratch_types=[
            pltpu.VMEM((KC,), f32),
            pltpu.VMEM((NCH, KC), i32),
            pltpu.VMEM((RPT,), f32),
            pltpu.VMEM_SHARED((NP_,), f32),
        ] + [pltpu.SemaphoreType.DMA] * NBUF,
    )(dst2)


CNB = 2
IST = 16
NST = NCH // IST


def _conv1_body(src_hbm, dst_hbm, h0p_hbm, dinv_hbm, s1_hbm, ca_hbm, *sc):
    idx_s, idx_d = sc[0], sc[1]
    rows = sc[2:2 + CNB]
    aval = sc[4:4 + CNB]
    acc_sh, cacc_sh = sc[6], sc[7]
    gsem = sc[8:8 + CNB]
    hsem = sc[10:10 + CNB]
    ssem = sc[12:12 + CNB]
    tsem = sc[14:14 + CNB]
    c = lax.axis_index("c")
    s = lax.axis_index("s")
    wid = c * 16 + s

    def zrow(i, _):
        def zcol(j, _):
            rows[0][i, pl.ds(j * 16, 16)] = jnp.zeros((16,), f32)
            return 0
        lax.fori_loop(0, HID // 16, zcol, 0)
        return 0
    lax.fori_loop(0, KC, zrow, 0)
    _fill(aval[0], KC, 0.0)

    def zacc(k, _):
        pltpu.sync_copy(rows[0], acc_sh.at[pl.ds(s * RPT + k * KC, KC)])
        pltpu.sync_copy(aval[0], cacc_sh.at[pl.ds(s * RPT + k * KC, KC)])
        return 0
    lax.fori_loop(0, RPT // KC, zacc, 0)
    plsc.subcore_barrier()

    def stage(st, _):
        base = wid * NCH + st * IST
        pltpu.sync_copy(src_hbm.at[pl.ds(base, IST)], idx_s)
        pltpu.sync_copy(dst_hbm.at[pl.ds(base, IST)], idx_d)

        def group(g, _):
            gd, hd = [], []
            for b in range(CNB):
                j = g * CNB + b
                gd.append(pltpu.async_copy(h0p_hbm.at[idx_s.at[j]], rows[b],
                                           gsem[b]))
                hd.append(pltpu.async_copy(dinv_hbm.at[idx_d.at[j]], aval[b],
                                           hsem[b]))
            sd, td = [], []
            for b in range(CNB):
                j = g * CNB + b
                gd[b].wait()
                sd.append(pltpu.async_copy(rows[b], acc_sh.at[idx_d.at[j]],
                                           ssem[b], add=True))
                hd[b].wait()
                td.append(pltpu.async_copy(aval[b], cacc_sh.at[idx_s.at[j]],
                                           tsem[b], add=True))
            for b in range(CNB):
                sd[b].wait()
                td[b].wait()
            return 0
        lax.fori_loop(0, IST // CNB, group, 0)
        return 0
    lax.fori_loop(0, NST, stage, 0)

    plsc.subcore_barrier()

    def cout(k, _):
        pltpu.sync_copy(acc_sh.at[pl.ds(s * RPT + k * KC, KC)], rows[0])
        pltpu.sync_copy(rows[0], s1_hbm.at[pl.ds(c * NP_ + s * RPT + k * KC, KC)])
        pltpu.sync_copy(cacc_sh.at[pl.ds(s * RPT + k * KC, KC)], aval[0])
        pltpu.sync_copy(aval[0], ca_hbm.at[pl.ds(c * NP_ + s * RPT + k * KC, KC)])
        return 0
    lax.fori_loop(0, RPT // KC, cout, 0)


def _conv1_kernel(src2, dst2, h0p, dinv):
    return pl.kernel(
        _conv1_body,
        out_type=[
            jax.ShapeDtypeStruct((2 * NP_, HID), f32),
            jax.ShapeDtypeStruct((2 * NP_,), f32),
        ],
        mesh=_sc_mesh(),
        scratch_types=[
            pltpu.VMEM((IST, KC), i32),
            pltpu.VMEM((IST, KC), i32),
        ] + [pltpu.VMEM((KC, HID), f32)] * CNB
          + [pltpu.VMEM((KC,), f32)] * CNB
          + [
            pltpu.VMEM_SHARED((NP_, HID), f32),
            pltpu.VMEM_SHARED((NP_,), f32),
        ] + [pltpu.SemaphoreType.DMA] * (4 * CNB),
    )(src2, dst2, h0p, dinv)


def _passb_body(src_hbm, dst_hbm, u_hbm, out_hbm, *sc):
    idx_s, idx_d = sc[0], sc[1]
    aval = sc[2:2 + NBUF]
    tmp_v, acc_sh = sc[6], sc[7]
    hsem = sc[8:12]
    tsem = sc[12:16]
    c = lax.axis_index("c")
    s = lax.axis_index("s")
    wid = c * 16 + s
    _fill(tmp_v, RPT, 0.0)
    pltpu.sync_copy(tmp_v, acc_sh.at[pl.ds(s * RPT, RPT)])
    pltpu.sync_copy(src_hbm.at[pl.ds(wid * NCH, NCH)], idx_s)
    pltpu.sync_copy(dst_hbm.at[pl.ds(wid * NCH, NCH)], idx_d)
    plsc.subcore_barrier()

    def group(g, _):
        hd = []
        for b in range(NBUF):
            j = g * NBUF + b
            hd.append(pltpu.async_copy(u_hbm.at[idx_d.at[j]], aval[b],
                                       hsem[b]))
        td = []
        for b in range(NBUF):
            j = g * NBUF + b
            hd[b].wait()
            td.append(pltpu.async_copy(aval[b], acc_sh.at[idx_s.at[j]],
                                       tsem[b], add=True))
        for b in range(NBUF):
            td[b].wait()
        return 0
    lax.fori_loop(0, NG, group, 0)

    plsc.subcore_barrier()
    pltpu.sync_copy(acc_sh.at[pl.ds(s * RPT, RPT)], tmp_v)
    pltpu.sync_copy(tmp_v, out_hbm.at[pl.ds(c * NP_ + s * RPT, RPT)])


def _passb_kernel(src2, dst2, u):
    return pl.kernel(
        _passb_body,
        out_type=jax.ShapeDtypeStruct((2 * NP_,), f32),
        mesh=_sc_mesh(),
        scratch_types=[
            pltpu.VMEM((NCH, KC), i32),
            pltpu.VMEM((NCH, KC), i32),
        ] + [pltpu.VMEM((KC,), f32)] * NBUF
          + [
            pltpu.VMEM((RPT,), f32),
            pltpu.VMEM_SHARED((NP_,), f32),
        ] + [pltpu.SemaphoreType.DMA] * (2 * NBUF),
    )(src2, dst2, u)


BR = 512
GRID = NP_ // BR


def _tc1_body(x_ref, w1_ref, dega_ref, degb_ref, h0p_ref, dinv_ref):
    deg = dega_ref[0] + degb_ref[0] + 1.0
    dinv = lax.rsqrt(deg)
    h = jnp.dot(x_ref[...], w1_ref[...], preferred_element_type=f32)
    h0p_ref[...] = h * dinv
    dinv_ref[...] = dinv


def _tc1(x, w1, degp):
    return pl.pallas_call(
        _tc1_body,
        grid=(GRID,),
        in_specs=[
            pl.BlockSpec((BR, HID), lambda i: (i, 0)),
            pl.BlockSpec((HID, HID), lambda i: (0, 0)),
            pl.BlockSpec((1, BR, 1), lambda i: (0, i, 0)),
            pl.BlockSpec((1, BR, 1), lambda i: (1, i, 0)),
        ],
        out_specs=[
            pl.BlockSpec((BR, HID), lambda i: (i, 0)),
            pl.BlockSpec((BR, 1), lambda i: (i, 0)),
        ],
        out_shape=[
            jax.ShapeDtypeStruct((NP_, HID), f32),
            jax.ShapeDtypeStruct((NP_, 1), f32),
        ],
    )(x, w1, degp, degp)


def _tc2_body(s1a_ref, s1b_ref, h0p_ref, dinv_ref, ca_ref, cb_ref, b1_ref,
              g1_ref, u_ref, w_ref):
    i = pl.program_id(0)
    rows = lax.broadcasted_iota(i32, (BR, 1), 0) + i * BR
    m = (rows < NN).astype(f32)
    dinv = dinv_ref[...]
    s1 = s1a_ref[0] + s1b_ref[0]
    g1 = jnp.maximum(dinv * (s1 + h0p_ref[...]) + b1_ref[...], 0.0)
    g1_ref[...] = g1 * m
    cc = ca_ref[0] + cb_ref[0]
    w = (cc + dinv) * dinv * m
    w_ref[...] = w
    u_ref[...] = w * dinv


def _tc2(s1p, h0p, dinv, cp, b1):
    return pl.pallas_call(
        _tc2_body,
        grid=(GRID,),
        in_specs=[
            pl.BlockSpec((1, BR, HID), lambda i: (0, i, 0)),
            pl.BlockSpec((1, BR, HID), lambda i: (1, i, 0)),
            pl.BlockSpec((BR, HID), lambda i: (i, 0)),
            pl.BlockSpec((BR, 1), lambda i: (i, 0)),
            pl.BlockSpec((1, BR, 1), lambda i: (0, i, 0)),
            pl.BlockSpec((1, BR, 1), lambda i: (1, i, 0)),
            pl.BlockSpec((1, HID), lambda i: (0, 0)),
        ],
        out_specs=[
            pl.BlockSpec((BR, HID), lambda i: (i, 0)),
            pl.BlockSpec((BR, 1), lambda i: (i, 0)),
            pl.BlockSpec((BR, 1), lambda i: (i, 0)),
        ],
        out_shape=[
            jax.ShapeDtypeStruct((NP_, HID), f32),
            jax.ShapeDtypeStruct((NP_, 1), f32),
            jax.ShapeDtypeStruct((NP_, 1), f32),
        ],
    )(s1p, s1p, h0p, dinv, cp, cp, b1)


def _tc3_body(g1_ref, u_ref, va_ref, vb_ref, dinv_ref, w_ref,
              rates_ref, we1_ref, be1_ref, we2_ref, be2_ref,
              w2a_ref, w2b_ref, b2_ref, w3_ref, b3_ref,
              wh_ref, bh_ref, wh2_ref, bh2_ref, wf_ref, bf_ref,
              out_ref, tacc, sacc):
    i = pl.program_id(0)

    @pl.when(i == 0)
    def _():
        tacc[...] = jnp.zeros((1, HID), f32)
        sacc[0] = 0.0
        sacc[1] = 0.0

    rows = lax.broadcasted_iota(i32, (BR, 1), 0) + i * BR
    m = (rows < NN).astype(f32)
    q = (va_ref[0] + vb_ref[0] + u_ref[...]) * dinv_ref[...] * m
    tacc[...] += jnp.sum(q * g1_ref[...], axis=0, keepdims=True)
    sacc[0] += jnp.sum(q)
    sacc[1] += jnp.sum(w_ref[...])

    @pl.when(i == pl.num_programs(0) - 1)
    def _():
        dot = functools.partial(jnp.dot, preferred_element_type=f32)
        r = jnp.maximum(dot(rates_ref[...], we1_ref[...]) + be1_ref[...], 0.0)
        r = dot(r, we2_ref[...]) + be2_ref[...]
        wtg2 = (dot(tacc[...], w2a_ref[...]) + sacc[0] * dot(r, w2b_ref[...])
                + sacc[1] * b2_ref[...])
        pool = dot(wtg2, w3_ref[...]) * (1.0 / NN) + b3_ref[...]
        z = jnp.maximum(dot(pool, wh_ref[...]) + bh_ref[...], 0.0)
        z = jnp.maximum(dot(z, wh2_ref[...]) + bh2_ref[...], 0.0)
        out_ref[...] = dot(z, wf_ref[...]) + bf_ref[...]


def _tc3(g1, u, vp, dinv, w, smalls):
    full = lambda a: pl.BlockSpec(a.shape, lambda i: tuple(0 for _ in a.shape))
    return pl.pallas_call(
        _tc3_body,
        grid=(GRID,),
        in_specs=[
            pl.BlockSpec((BR, HID), lambda i: (i, 0)),
            pl.BlockSpec((BR, 1), lambda i: (i, 0)),
            pl.BlockSpec((1, BR, 1), lambda i: (0, i, 0)),
            pl.BlockSpec((1, BR, 1), lambda i: (1, i, 0)),
            pl.BlockSpec((BR, 1), lambda i: (i, 0)),
            pl.BlockSpec((BR, 1), lambda i: (i, 0)),
        ] + [full(a) for a in smalls],
        out_specs=pl.BlockSpec((1, 2), lambda i: (0, 0)),
        out_shape=jax.ShapeDtypeStruct((1, 2), f32),
        scratch_shapes=[pltpu.VMEM((1, HID), f32), pltpu.SMEM((2,), f32)],
    )(g1, u, vp, vp, dinv, w, *smalls)


def kernel(graph, edge_index, rates, W1, b1, We1, be1, We2, be2, W2, b2,
           W3, b3, Wh, bh, Wh2, bh2, Wf, bf):
    pad = EPAD - EE
    padi = NN + jnp.arange(pad, dtype=i32) % (NP_ - NN)
    src = jnp.concatenate([edge_index[0], padi])
    dst = jnp.concatenate([edge_index[1], padi])
    src2 = src.reshape(NWK * NCH, KC)
    dst2 = dst.reshape(NWK * NCH, KC)
    x = jnp.pad(graph, ((0, NP_ - NN), (0, 0)))

    degp = _deg_kernel(dst2)
    h0p, dinv = _tc1(x, W1, degp.reshape(2, NP_, 1))
    s1p, cp = _conv1_kernel(src2, dst2, h0p, dinv.reshape(NP_))
    g1, u, w = _tc2(s1p.reshape(2, NP_, HID), h0p, dinv,
                    cp.reshape(2, NP_, 1), b1.reshape(1, HID))
    vp = _passb_kernel(src2, dst2, u.reshape(NP_))
    smalls = (rates.reshape(1, 16), We1, be1.reshape(1, 8), We2,
              be2.reshape(1, HID), W2[:HID], W2[HID:], b2.reshape(1, HID),
              W3, b3.reshape(1, HID), Wh, bh.reshape(1, 2 * HID), Wh2,
              bh2.reshape(1, HID), Wf, bf.reshape(1, 2))
    return _tc3(g1, u, vp.reshape(2, NP_, 1), dinv, w, smalls)
